# baseline (speedup 1.0000x reference)
import functools

import jax
import jax.numpy as jnp
from jax import lax
from jax.experimental import pallas as pl
from jax.experimental.pallas import tpu as pltpu
from jax.experimental.pallas import tpu_sc as plsc

_N = 10000
_NP = 10240
_E = 320000
_NC = 2
_NS = 16
_NW = _NC * _NS
_CHUNK = 128
_CPW = 80
_EP = _NW * _CPW * _CHUNK
_RPS = _NP // _NS


def _make_sc_aggregate(dpad, nbuf, idx_groups):
  mesh = plsc.VectorSubcoreMesh(core_axis_name="c", subcore_axis_name="s")
  bpg = _CPW // idx_groups

  @functools.partial(
      pl.kernel,
      mesh=mesh,
      compiler_params=pltpu.CompilerParams(use_tc_tiling_on_sc=False),
      out_type=jax.ShapeDtypeStruct((_NC, _NP, dpad), jnp.float32),
      scratch_types=(
          [pltpu.VMEM((bpg, _CHUNK), jnp.int32),
           pltpu.VMEM((bpg, _CHUNK), jnp.int32)]
          + [pltpu.VMEM((_CHUNK, dpad), jnp.float32) for _ in range(nbuf)]
          + [pltpu.VMEM_SHARED((_NP, dpad), jnp.float32)]
          + [pltpu.SemaphoreType.DMA for _ in range(2 * nbuf)]
      ),
  )
  def agg(table_hbm, src_hbm, dst_hbm, z_hbm, out_hbm, srcb, dstb, *rest):
    rbufs = rest[:nbuf]
    acc_sh = rest[nbuf]
    sgs = rest[nbuf + 1:2 * nbuf + 1]
    sss = rest[2 * nbuf + 1:]
    c = lax.axis_index("c")
    s = lax.axis_index("s")
    wid = c * _NS + s

    def g_desc(k, b):
      return pltpu.make_async_copy(table_hbm.at[srcb.at[k]], rbufs[b], sgs[b])

    def s_desc(k, b):
      return pltpu.make_async_copy(rbufs[b], acc_sh.at[dstb.at[k]], sss[b])

    def load_idx(g, sem_a, sem_b):
      a = pltpu.make_async_copy(
          src_hbm.at[pl.ds(wid * _CPW + g * bpg, bpg)], srcb, sem_a)
      bb = pltpu.make_async_copy(
          dst_hbm.at[pl.ds(wid * _CPW + g * bpg, bpg)], dstb, sem_b)
      a.start()
      bb.start()
      a.wait()
      bb.wait()

    idx_src = pltpu.make_async_copy(
        src_hbm.at[pl.ds(wid * _CPW, bpg)], srcb, sgs[0])
    idx_dst = pltpu.make_async_copy(
        dst_hbm.at[pl.ds(wid * _CPW, bpg)], dstb, sgs[1 % nbuf])
    idx_src.start()
    idx_dst.start()
    pltpu.sync_copy(z_hbm, acc_sh.at[pl.ds(s * _RPS, _RPS)])
    idx_src.wait()
    idx_dst.wait()
    plsc.subcore_barrier()

    for g in range(idx_groups):
      if g > 0:
        load_idx(g, sgs[0], sgs[1 % nbuf])
      for b in range(nbuf):
        g_desc(b, b).start()

      @pl.loop(0, bpg // nbuf - 1)
      def _(j):
        k = j * nbuf
        for b in range(nbuf):
          g_desc(k + b, b).wait()
          s_desc(k + b, b).start(add=True)
        for b in range(nbuf):
          s_desc(k + b, b).wait()
          g_desc(k + nbuf + b, b).start()

      base = bpg - nbuf
      for b in range(nbuf):
        g_desc(base + b, b).wait()
        s_desc(base + b, b).start(add=True)
      for b in range(nbuf):
        s_desc(base + b, b).wait()

    plsc.subcore_barrier()

    pltpu.sync_copy(acc_sh.at[pl.ds(s * _RPS, _RPS)],
                    out_hbm.at[c].at[pl.ds(s * _RPS, _RPS)])

  return agg


def _lrelu(x):
  return jnp.where(x >= 0, x, 0.01 * x)


def _bn(x):
  m = jnp.mean(x, axis=0, keepdims=True)
  v = jnp.mean((x - m) * (x - m), axis=0, keepdims=True)
  return (x - m) * lax.rsqrt(v + 1e-5)


def _dot(a, b):
  return jnp.dot(a, b, preferred_element_type=jnp.float32)


def _tc_layer1a(aggp, cntp, x, Wl1, bl1, Wr1):
  def body(aggp_ref, cntp_ref, x_ref, wl_ref, bl_ref, wr_ref, z_ref, inv_ref):
    cnt = cntp_ref[0][:_N, 0:1] + cntp_ref[1][:_N, 0:1]
    inv = 1.0 / jnp.maximum(cnt, 1.0)
    mean = (aggp_ref[0][:_N] + aggp_ref[1][:_N]) * inv
    z = _dot(mean, wl_ref[...]) + bl_ref[...][None, :] + _dot(x_ref[...], wr_ref[...])
    z_ref[...] = _lrelu(z)
    inv_ref[...] = inv

  return pl.pallas_call(
      body,
      out_shape=[
          jax.ShapeDtypeStruct((_N, 320), jnp.float32),
          jax.ShapeDtypeStruct((_N, 1), jnp.float32),
      ],
  )(aggp, cntp, x, Wl1, bl1, Wr1)


def _tc_layer1b(z, Wl2, Wr2):
  def body(z_ref, wl_ref, wr_ref, ha_ref, hb_ref, xw_ref):
    y = _bn(z_ref[...])
    h = _dot(y, wl_ref[...])
    ha_ref[...] = h[:, :128]
    hb_ref[...] = jnp.pad(h[:, 128:], ((0, 0), (0, 12)))
    xw_ref[...] = _dot(y, wr_ref[...])

  return pl.pallas_call(
      body,
      out_shape=[
          jax.ShapeDtypeStruct((_N, 128), jnp.float32),
          jax.ShapeDtypeStruct((_N, 64), jnp.float32),
          jax.ShapeDtypeStruct((_N, 180), jnp.float32),
      ],
  )(z, Wl2, Wr2)


def _tc_layer2(aggpa, aggpb, xw, inv, bl, Wl_next, Wr_next):
  def body(aggpa_ref, aggpb_ref, xw_ref, inv_ref, bl_ref, wln_ref, wrn_ref,
           hp_ref, xwn_ref):
    agg = jnp.concatenate(
        [aggpa_ref[0][:_N] + aggpa_ref[1][:_N],
         aggpb_ref[0][:_N, :52] + aggpb_ref[1][:_N, :52]], axis=1)
    y = agg * inv_ref[...] + bl_ref[...][None, :] + xw_ref[...]
    y = _bn(_lrelu(y))
    h = _dot(y, wln_ref[...])
    hp_ref[...] = jnp.pad(h, ((0, 0), (0, 6)))
    xwn_ref[...] = _dot(y, wrn_ref[...])

  return pl.pallas_call(
      body,
      out_shape=[
          jax.ShapeDtypeStruct((_N, 96), jnp.float32),
          jax.ShapeDtypeStruct((_N, 90), jnp.float32),
      ],
  )(aggpa, aggpb, xw, inv, bl, Wl_next, Wr_next)


def _tc_layer3(aggp, xw, inv, bl, Wl_next, Wr_next):
  def body(aggp_ref, xw_ref, inv_ref, bl_ref, wln_ref, wrn_ref,
           hp_ref, xwn_ref):
    agg = aggp_ref[0][:_N, :90] + aggp_ref[1][:_N, :90]
    y = agg * inv_ref[...] + bl_ref[...][None, :] + xw_ref[...]
    y = _bn(_lrelu(y))
    h = _dot(y, wln_ref[...])
    hp_ref[...] = jnp.pad(h, ((0, 0), (0, 14)))
    xwn_ref[...] = _dot(y, wrn_ref[...])

  return pl.pallas_call(
      body,
      out_shape=[
          jax.ShapeDtypeStruct((_N, 64), jnp.float32),
          jax.ShapeDtypeStruct((_N, 50), jnp.float32),
      ],
  )(aggp, xw, inv, bl, Wl_next, Wr_next)


def _tc_layer4(aggp, xw, inv, bl4, fW1, fb1, fW2, fb2, fW3, fb3):
  blen = _N // 16

  def body(aggp_ref, xw_ref, inv_ref, bl_ref,
           fw1_ref, fb1_ref, fw2_ref, fb2_ref, fw3_ref, fb3_ref, out_ref):
    agg = aggp_ref[0][:_N, :50] + aggp_ref[1][:_N, :50]
    y = agg * inv_ref[...] + bl_ref[...][None, :] + xw_ref[...]
    y = _bn(_lrelu(y))
    col = lax.broadcasted_iota(jnp.int32, (16, _N), 1) // blen
    row = lax.broadcasted_iota(jnp.int32, (16, _N), 0)
    sel = (col == row).astype(jnp.float32)
    p = _dot(sel, y)
    p = _dot(p, fw1_ref[...]) + fb1_ref[...][None, :]
    p = _dot(p, fw2_ref[...]) + fb2_ref[...][None, :]
    p = _dot(p, fw3_ref[...]) + fb3_ref[...][None, :]
    out_ref[...] = p

  return pl.pallas_call(
      body,
      out_shape=jax.ShapeDtypeStruct((16, 1), jnp.float32),
  )(aggp, xw, inv, bl4, fW1, fb1, fW2, fb2, fW3, fb3)


_agg16 = _make_sc_aggregate(16, 1, 1)
_agg128 = _make_sc_aggregate(128, 1, 1)
_agg96 = _make_sc_aggregate(96, 1, 1)
_agg64 = _make_sc_aggregate(64, 1, 1)


def kernel(x_in, edge_index, Wl1, bl1, Wr1, Wl2, bl2, Wr2, Wl3, bl3, Wr3,
           Wl4, bl4, Wr4, fW1, fb1, fW2, fb2, fW3, fb3):
  src = jnp.reshape(
      jnp.concatenate([edge_index[0],
                       jnp.zeros((_EP - _E,), jnp.int32)]),
      (_EP // _CHUNK, _CHUNK))
  pad_dst = _N + jnp.arange(_EP - _E, dtype=jnp.int32) % (_NP - _N)
  dst = jnp.reshape(
      jnp.concatenate([edge_index[1], pad_dst]),
      (_EP // _CHUNK, _CHUNK))

  ac = _agg16(jnp.ones((_N, 16), jnp.float32), src, dst,
              jnp.zeros((_RPS, 16), jnp.float32))
  a1 = _agg128(x_in, src, dst, jnp.zeros((_RPS, 128), jnp.float32))
  z1, inv = _tc_layer1a(a1, ac, x_in, Wl1, bl1, Wr1)
  h2a, h2b, xw2 = _tc_layer1b(z1, Wl2, Wr2)

  a2a = _agg128(h2a, src, dst, jnp.zeros((_RPS, 128), jnp.float32))
  a2b = _agg64(h2b, src, dst, jnp.zeros((_RPS, 64), jnp.float32))
  h3p, xw3 = _tc_layer2(a2a, a2b, xw2, inv, bl2, Wl3, Wr3)

  a3 = _agg96(h3p, src, dst, jnp.zeros((_RPS, 96), jnp.float32))
  h4p, xw4 = _tc_layer3(a3, xw3, inv, bl3, Wl4, Wr4)

  a4 = _agg64(h4p, src, dst, jnp.zeros((_RPS, 64), jnp.float32))
  return _tc_layer4(a4, xw4, inv, bl4, fW1, fb1, fW2, fb2, fW3, fb3)


# speedup vs baseline: 1.2109x; 1.2109x over previous
import functools

import jax
import jax.numpy as jnp
from jax import lax
from jax.experimental import pallas as pl
from jax.experimental.pallas import tpu as pltpu
from jax.experimental.pallas import tpu_sc as plsc

_N = 10000
_NP = 10240
_E = 320000
_NC = 2
_NS = 16
_NW = _NC * _NS
_CHUNK = 128
_CPW0 = 120
_CPW1 = 40
_BPG = 40
_EP = _NS * (_CPW0 + _CPW1) * _CHUNK
_RPS = _NP // _NS


def _make_sc_aggregate(dpad, nbuf):
  mesh = plsc.VectorSubcoreMesh(core_axis_name="c", subcore_axis_name="s")

  @functools.partial(
      pl.kernel,
      mesh=mesh,
      compiler_params=pltpu.CompilerParams(use_tc_tiling_on_sc=False),
      out_type=jax.ShapeDtypeStruct((_NC, _NP, dpad), jnp.float32),
      scratch_types=(
          [pltpu.VMEM((_BPG, _CHUNK), jnp.int32),
           pltpu.VMEM((_BPG, _CHUNK), jnp.int32)]
          + [pltpu.VMEM((_CHUNK, dpad), jnp.float32) for _ in range(nbuf)]
          + [pltpu.VMEM_SHARED((_NP, dpad), jnp.float32)]
          + [pltpu.SemaphoreType.DMA for _ in range(2 * nbuf)]
      ),
  )
  def agg(table_hbm, src_hbm, dst_hbm, z_hbm, out_hbm, srcb, dstb, *rest):
    rbufs = rest[:nbuf]
    acc_sh = rest[nbuf]
    sgs = rest[nbuf + 1:2 * nbuf + 1]
    sss = rest[2 * nbuf + 1:]
    c = lax.axis_index("c")
    s = lax.axis_index("s")
    base = jnp.where(c == 0, s * _CPW0, _NS * _CPW0 + s * _CPW1)

    def g_desc(k, b):
      return pltpu.make_async_copy(table_hbm.at[srcb.at[k]], rbufs[b], sgs[b])

    def s_desc(k, b):
      return pltpu.make_async_copy(rbufs[b], acc_sh.at[dstb.at[k]], sss[b])

    def load_idx_start(g):
      a = pltpu.make_async_copy(
          src_hbm.at[pl.ds(base + g * _BPG, _BPG)], srcb, sgs[0])
      bb = pltpu.make_async_copy(
          dst_hbm.at[pl.ds(base + g * _BPG, _BPG)], dstb, sgs[1 % nbuf])
      a.start()
      bb.start()
      return a, bb

    def pipe_block():
      for b in range(nbuf):
        g_desc(b, b).start()

      @pl.loop(0, _BPG // nbuf - 1)
      def _(j):
        k = j * nbuf
        for b in range(nbuf):
          g_desc(k + b, b).wait()
          s_desc(k + b, b).start(add=True)
        for b in range(nbuf):
          s_desc(k + b, b).wait()
          g_desc(k + nbuf + b, b).start()

      tail = _BPG - nbuf
      for b in range(nbuf):
        g_desc(tail + b, b).wait()
        s_desc(tail + b, b).start(add=True)
      for b in range(nbuf):
        s_desc(tail + b, b).wait()

    a, bb = load_idx_start(0)
    pltpu.sync_copy(z_hbm, acc_sh.at[pl.ds(s * _RPS, _RPS)])
    a.wait()
    bb.wait()
    plsc.subcore_barrier()

    pipe_block()

    @pl.when(c == 0)
    def _():
      for g in range(1, _CPW0 // _BPG):
        a, bb = load_idx_start(g)
        a.wait()
        bb.wait()
        pipe_block()

    plsc.subcore_barrier()

    pltpu.sync_copy(acc_sh.at[pl.ds(s * _RPS, _RPS)],
                    out_hbm.at[c].at[pl.ds(s * _RPS, _RPS)])

  return agg


def _lrelu(x):
  return jnp.where(x >= 0, x, 0.01 * x)


def _bn(x):
  m = jnp.mean(x, axis=0, keepdims=True)
  v = jnp.mean((x - m) * (x - m), axis=0, keepdims=True)
  return (x - m) * lax.rsqrt(v + 1e-5)


def _dot(a, b):
  return jnp.dot(a, b, preferred_element_type=jnp.float32)


def _tc_layer1a(aggp, cntp, x, Wl1, bl1, Wr1):
  def body(aggp_ref, cntp_ref, x_ref, wl_ref, bl_ref, wr_ref, z_ref, inv_ref):
    cnt = cntp_ref[0][:_N, 0:1] + cntp_ref[1][:_N, 0:1]
    inv = 1.0 / jnp.maximum(cnt, 1.0)
    mean = (aggp_ref[0][:_N] + aggp_ref[1][:_N]) * inv
    z = _dot(mean, wl_ref[...]) + bl_ref[...][None, :] + _dot(x_ref[...], wr_ref[...])
    z_ref[...] = _lrelu(z)
    inv_ref[...] = inv

  return pl.pallas_call(
      body,
      out_shape=[
          jax.ShapeDtypeStruct((_N, 320), jnp.float32),
          jax.ShapeDtypeStruct((_N, 1), jnp.float32),
      ],
  )(aggp, cntp, x, Wl1, bl1, Wr1)


def _tc_layer1b(z, Wl2, Wr2):
  def body(z_ref, wl_ref, wr_ref, ha_ref, hb_ref, xw_ref):
    y = _bn(z_ref[...])
    h = _dot(y, wl_ref[...])
    ha_ref[...] = h[:, :128]
    hb_ref[...] = jnp.pad(h[:, 128:], ((0, 0), (0, 12)))
    xw_ref[...] = _dot(y, wr_ref[...])

  return pl.pallas_call(
      body,
      out_shape=[
          jax.ShapeDtypeStruct((_N, 128), jnp.float32),
          jax.ShapeDtypeStruct((_N, 64), jnp.float32),
          jax.ShapeDtypeStruct((_N, 180), jnp.float32),
      ],
  )(z, Wl2, Wr2)


def _tc_layer2(aggpa, aggpb, xw, inv, bl, Wl_next, Wr_next):
  def body(aggpa_ref, aggpb_ref, xw_ref, inv_ref, bl_ref, wln_ref, wrn_ref,
           hp_ref, xwn_ref):
    agg = jnp.concatenate(
        [aggpa_ref[0][:_N] + aggpa_ref[1][:_N],
         aggpb_ref[0][:_N, :52] + aggpb_ref[1][:_N, :52]], axis=1)
    y = agg * inv_ref[...] + bl_ref[...][None, :] + xw_ref[...]
    y = _bn(_lrelu(y))
    h = _dot(y, wln_ref[...])
    hp_ref[...] = jnp.pad(h, ((0, 0), (0, 6)))
    xwn_ref[...] = _dot(y, wrn_ref[...])

  return pl.pallas_call(
      body,
      out_shape=[
          jax.ShapeDtypeStruct((_N, 96), jnp.float32),
          jax.ShapeDtypeStruct((_N, 90), jnp.float32),
      ],
  )(aggpa, aggpb, xw, inv, bl, Wl_next, Wr_next)


def _tc_layer3(aggp, xw, inv, bl, Wl_next, Wr_next):
  def body(aggp_ref, xw_ref, inv_ref, bl_ref, wln_ref, wrn_ref,
           hp_ref, xwn_ref):
    agg = aggp_ref[0][:_N, :90] + aggp_ref[1][:_N, :90]
    y = agg * inv_ref[...] + bl_ref[...][None, :] + xw_ref[...]
    y = _bn(_lrelu(y))
    h = _dot(y, wln_ref[...])
    hp_ref[...] = jnp.pad(h, ((0, 0), (0, 14)))
    xwn_ref[...] = _dot(y, wrn_ref[...])

  return pl.pallas_call(
      body,
      out_shape=[
          jax.ShapeDtypeStruct((_N, 64), jnp.float32),
          jax.ShapeDtypeStruct((_N, 50), jnp.float32),
      ],
  )(aggp, xw, inv, bl, Wl_next, Wr_next)


def _tc_layer4(aggp, xw, inv, bl4, fW1, fb1, fW2, fb2, fW3, fb3):
  blen = _N // 16

  def body(aggp_ref, xw_ref, inv_ref, bl_ref,
           fw1_ref, fb1_ref, fw2_ref, fb2_ref, fw3_ref, fb3_ref, out_ref):
    agg = aggp_ref[0][:_N, :50] + aggp_ref[1][:_N, :50]
    y = agg * inv_ref[...] + bl_ref[...][None, :] + xw_ref[...]
    y = _bn(_lrelu(y))
    col = lax.broadcasted_iota(jnp.int32, (16, _N), 1) // blen
    row = lax.broadcasted_iota(jnp.int32, (16, _N), 0)
    sel = (col == row).astype(jnp.float32)
    p = _dot(sel, y)
    p = _dot(p, fw1_ref[...]) + fb1_ref[...][None, :]
    p = _dot(p, fw2_ref[...]) + fb2_ref[...][None, :]
    p = _dot(p, fw3_ref[...]) + fb3_ref[...][None, :]
    out_ref[...] = p

  return pl.pallas_call(
      body,
      out_shape=jax.ShapeDtypeStruct((16, 1), jnp.float32),
  )(aggp, xw, inv, bl4, fW1, fb1, fW2, fb2, fW3, fb3)


_agg16 = _make_sc_aggregate(16, 4)
_agg128 = _make_sc_aggregate(128, 2)
_agg96 = _make_sc_aggregate(96, 4)
_agg64 = _make_sc_aggregate(64, 4)


def kernel(x_in, edge_index, Wl1, bl1, Wr1, Wl2, bl2, Wr2, Wl3, bl3, Wr3,
           Wl4, bl4, Wr4, fW1, fb1, fW2, fb2, fW3, fb3):
  src = jnp.reshape(
      jnp.concatenate([edge_index[0],
                       jnp.zeros((_EP - _E,), jnp.int32)]),
      (_EP // _CHUNK, _CHUNK))
  pad_dst = _N + jnp.arange(_EP - _E, dtype=jnp.int32) % (_NP - _N)
  dst = jnp.reshape(
      jnp.concatenate([edge_index[1], pad_dst]),
      (_EP // _CHUNK, _CHUNK))

  ac = _agg16(jnp.ones((_N, 16), jnp.float32), src, dst,
              jnp.zeros((_RPS, 16), jnp.float32))
  a1 = _agg128(x_in, src, dst, jnp.zeros((_RPS, 128), jnp.float32))
  z1, inv = _tc_layer1a(a1, ac, x_in, Wl1, bl1, Wr1)
  h2a, h2b, xw2 = _tc_layer1b(z1, Wl2, Wr2)

  a2a = _agg128(h2a, src, dst, jnp.zeros((_RPS, 128), jnp.float32))
  a2b = _agg64(h2b, src, dst, jnp.zeros((_RPS, 64), jnp.float32))
  h3p, xw3 = _tc_layer2(a2a, a2b, xw2, inv, bl2, Wl3, Wr3)

  a3 = _agg96(h3p, src, dst, jnp.zeros((_RPS, 96), jnp.float32))
  h4p, xw4 = _tc_layer3(a3, xw3, inv, bl3, Wl4, Wr4)

  a4 = _agg64(h4p, src, dst, jnp.zeros((_RPS, 64), jnp.float32))
  return _tc_layer4(a4, xw4, inv, bl4, fW1, fb1, fW2, fb2, fW3, fb3)


# speedup vs baseline: 1.2729x; 1.0512x over previous
import functools

import jax
import jax.numpy as jnp
from jax import lax
from jax.experimental import pallas as pl
from jax.experimental.pallas import tpu as pltpu
from jax.experimental.pallas import tpu_sc as plsc

_N = 10000
_NP = 10240
_E = 320000
_NC = 2
_NS = 16
_NW = _NC * _NS
_CHUNK = 128
_CPW0 = 120
_CPW1 = 40
_BPG = 40
_EP = _NS * (_CPW0 + _CPW1) * _CHUNK
_RPS = _NP // _NS


def _make_sc_aggregate(dpad, nbuf):
  mesh = plsc.VectorSubcoreMesh(core_axis_name="c", subcore_axis_name="s")

  @functools.partial(
      pl.kernel,
      mesh=mesh,
      compiler_params=pltpu.CompilerParams(use_tc_tiling_on_sc=False),
      out_type=jax.ShapeDtypeStruct((_NC, _NP, dpad), jnp.float32),
      scratch_types=(
          [pltpu.VMEM((_BPG, _CHUNK), jnp.int32),
           pltpu.VMEM((_BPG, _CHUNK), jnp.int32)]
          + [pltpu.VMEM((_CHUNK, dpad), jnp.float32) for _ in range(nbuf)]
          + [pltpu.VMEM_SHARED((_NP, dpad), jnp.float32)]
          + [pltpu.SemaphoreType.DMA for _ in range(2 * nbuf)]
      ),
  )
  def agg(table_hbm, src_hbm, dst_hbm, out_hbm, srcb, dstb, *rest):
    rbufs = rest[:nbuf]
    acc_sh = rest[nbuf]
    sgs = rest[nbuf + 1:2 * nbuf + 1]
    sss = rest[2 * nbuf + 1:]
    c = lax.axis_index("c")
    s = lax.axis_index("s")
    base = jnp.where(c == 0, s * _CPW0, _NS * _CPW0 + s * _CPW1)

    def g_desc(k, b):
      return pltpu.make_async_copy(table_hbm.at[srcb.at[k]], rbufs[b], sgs[b])

    def s_desc(k, b):
      return pltpu.make_async_copy(rbufs[b], acc_sh.at[dstb.at[k]], sss[b])

    def load_idx_start(g):
      a = pltpu.make_async_copy(
          src_hbm.at[pl.ds(base + g * _BPG, _BPG)], srcb, sgs[0])
      bb = pltpu.make_async_copy(
          dst_hbm.at[pl.ds(base + g * _BPG, _BPG)], dstb, sgs[1 % nbuf])
      a.start()
      bb.start()
      return a, bb

    def pipe_block():
      for b in range(nbuf):
        g_desc(b, b).start()

      @pl.loop(0, _BPG // nbuf - 1)
      def _(j):
        k = j * nbuf
        for b in range(nbuf):
          g_desc(k + b, b).wait()
          s_desc(k + b, b).start(add=True)
        for b in range(nbuf):
          s_desc(k + b, b).wait()
          g_desc(k + nbuf + b, b).start()

      tail = _BPG - nbuf
      for b in range(nbuf):
        g_desc(tail + b, b).wait()
        s_desc(tail + b, b).start(add=True)
      for b in range(nbuf):
        s_desc(tail + b, b).wait()

    a, bb = load_idx_start(0)

    @pl.loop(0, _CHUNK)
    def _(i):
      @pl.loop(0, dpad, step=16)
      def _(j):
        rbufs[0][i, pl.ds(j, 16)] = jnp.zeros((16,), jnp.float32)

    for r in range(_RPS // _CHUNK):
      pltpu.sync_copy(rbufs[0],
                      acc_sh.at[pl.ds(s * _RPS + r * _CHUNK, _CHUNK)])
    a.wait()
    bb.wait()
    plsc.subcore_barrier()

    pipe_block()

    @pl.when(c == 0)
    def _():
      for g in range(1, _CPW0 // _BPG):
        a, bb = load_idx_start(g)
        a.wait()
        bb.wait()
        pipe_block()

    plsc.subcore_barrier()

    pltpu.sync_copy(acc_sh.at[pl.ds(s * _RPS, _RPS)],
                    out_hbm.at[c].at[pl.ds(s * _RPS, _RPS)])

  return agg


def _lrelu(x):
  return jnp.where(x >= 0, x, 0.01 * x)


def _bn(x):
  m = jnp.mean(x, axis=0, keepdims=True)
  v = jnp.mean((x - m) * (x - m), axis=0, keepdims=True)
  return (x - m) * lax.rsqrt(v + 1e-5)


def _dot(a, b):
  return jnp.dot(a, b, preferred_element_type=jnp.float32)


def _tc_layer1a(aggp, cntp, x, Wl1, bl1, Wr1):
  def body(aggp_ref, cntp_ref, x_ref, wl_ref, bl_ref, wr_ref, z_ref, inv_ref):
    cnt = cntp_ref[0][:_N, 0:1] + cntp_ref[1][:_N, 0:1]
    inv = 1.0 / jnp.maximum(cnt, 1.0)
    mean = (aggp_ref[0][:_N] + aggp_ref[1][:_N]) * inv
    z = _dot(mean, wl_ref[...]) + bl_ref[...][None, :] + _dot(x_ref[...], wr_ref[...])
    z_ref[...] = _lrelu(z)
    inv_ref[...] = inv

  return pl.pallas_call(
      body,
      out_shape=[
          jax.ShapeDtypeStruct((_N, 320), jnp.float32),
          jax.ShapeDtypeStruct((_N, 1), jnp.float32),
      ],
  )(aggp, cntp, x, Wl1, bl1, Wr1)


def _tc_layer1b(z, Wl2, Wr2):
  def body(z_ref, wl_ref, wr_ref, ha_ref, hb_ref, xw_ref):
    y = _bn(z_ref[...])
    h = _dot(y, wl_ref[...])
    ha_ref[...] = h[:, :128]
    hb_ref[...] = jnp.pad(h[:, 128:], ((0, 0), (0, 12)))
    xw_ref[...] = _dot(y, wr_ref[...])

  return pl.pallas_call(
      body,
      out_shape=[
          jax.ShapeDtypeStruct((_N, 128), jnp.float32),
          jax.ShapeDtypeStruct((_N, 64), jnp.float32),
          jax.ShapeDtypeStruct((_N, 180), jnp.float32),
      ],
  )(z, Wl2, Wr2)


def _tc_layer2(aggpa, aggpb, xw, inv, bl, Wl_next, Wr_next):
  def body(aggpa_ref, aggpb_ref, xw_ref, inv_ref, bl_ref, wln_ref, wrn_ref,
           hp_ref, xwn_ref):
    agg = jnp.concatenate(
        [aggpa_ref[0][:_N] + aggpa_ref[1][:_N],
         aggpb_ref[0][:_N, :52] + aggpb_ref[1][:_N, :52]], axis=1)
    y = agg * inv_ref[...] + bl_ref[...][None, :] + xw_ref[...]
    y = _bn(_lrelu(y))
    h = _dot(y, wln_ref[...])
    hp_ref[...] = jnp.pad(h, ((0, 0), (0, 6)))
    xwn_ref[...] = _dot(y, wrn_ref[...])

  return pl.pallas_call(
      body,
      out_shape=[
          jax.ShapeDtypeStruct((_N, 96), jnp.float32),
          jax.ShapeDtypeStruct((_N, 90), jnp.float32),
      ],
  )(aggpa, aggpb, xw, inv, bl, Wl_next, Wr_next)


def _tc_layer3(aggp, xw, inv, bl, Wl_next, Wr_next):
  def body(aggp_ref, xw_ref, inv_ref, bl_ref, wln_ref, wrn_ref,
           hp_ref, xwn_ref):
    agg = aggp_ref[0][:_N, :90] + aggp_ref[1][:_N, :90]
    y = agg * inv_ref[...] + bl_ref[...][None, :] + xw_ref[...]
    y = _bn(_lrelu(y))
    h = _dot(y, wln_ref[...])
    hp_ref[...] = jnp.pad(h, ((0, 0), (0, 14)))
    xwn_ref[...] = _dot(y, wrn_ref[...])

  return pl.pallas_call(
      body,
      out_shape=[
          jax.ShapeDtypeStruct((_N, 64), jnp.float32),
          jax.ShapeDtypeStruct((_N, 50), jnp.float32),
      ],
  )(aggp, xw, inv, bl, Wl_next, Wr_next)


def _tc_layer4(aggp, xw, inv, bl4, fW1, fb1, fW2, fb2, fW3, fb3):
  blen = _N // 16

  def body(aggp_ref, xw_ref, inv_ref, bl_ref,
           fw1_ref, fb1_ref, fw2_ref, fb2_ref, fw3_ref, fb3_ref, out_ref):
    agg = aggp_ref[0][:_N, :50] + aggp_ref[1][:_N, :50]
    y = agg * inv_ref[...] + bl_ref[...][None, :] + xw_ref[...]
    y = _bn(_lrelu(y))
    col = lax.broadcasted_iota(jnp.int32, (16, _N), 1) // blen
    row = lax.broadcasted_iota(jnp.int32, (16, _N), 0)
    sel = (col == row).astype(jnp.float32)
    p = _dot(sel, y)
    p = _dot(p, fw1_ref[...]) + fb1_ref[...][None, :]
    p = _dot(p, fw2_ref[...]) + fb2_ref[...][None, :]
    p = _dot(p, fw3_ref[...]) + fb3_ref[...][None, :]
    out_ref[...] = p

  return pl.pallas_call(
      body,
      out_shape=jax.ShapeDtypeStruct((16, 1), jnp.float32),
  )(aggp, xw, inv, bl4, fW1, fb1, fW2, fb2, fW3, fb3)


_agg16 = _make_sc_aggregate(16, 4)
_agg128 = _make_sc_aggregate(128, 2)
_agg96 = _make_sc_aggregate(96, 4)
_agg64 = _make_sc_aggregate(64, 4)


def kernel(x_in, edge_index, Wl1, bl1, Wr1, Wl2, bl2, Wr2, Wl3, bl3, Wr3,
           Wl4, bl4, Wr4, fW1, fb1, fW2, fb2, fW3, fb3):
  src = jnp.reshape(
      jnp.concatenate([edge_index[0],
                       jnp.zeros((_EP - _E,), jnp.int32)]),
      (_EP // _CHUNK, _CHUNK))
  pad_dst = _N + jnp.arange(_EP - _E, dtype=jnp.int32) % (_NP - _N)
  dst = jnp.reshape(
      jnp.concatenate([edge_index[1], pad_dst]),
      (_EP // _CHUNK, _CHUNK))

  ac = _agg16(jnp.ones((_N, 16), jnp.float32), src, dst)
  a1 = _agg128(x_in, src, dst)
  z1, inv = _tc_layer1a(a1, ac, x_in, Wl1, bl1, Wr1)
  h2a, h2b, xw2 = _tc_layer1b(z1, Wl2, Wr2)

  a2a = _agg128(h2a, src, dst)
  a2b = _agg64(h2b, src, dst)
  h3p, xw3 = _tc_layer2(a2a, a2b, xw2, inv, bl2, Wl3, Wr3)

  a3 = _agg96(h3p, src, dst)
  h4p, xw4 = _tc_layer3(a3, xw3, inv, bl3, Wl4, Wr4)

  a4 = _agg64(h4p, src, dst)
  return _tc_layer4(a4, xw4, inv, bl4, fW1, fb1, fW2, fb2, fW3, fb3)
